# baseline (speedup 1.0000x reference)
import functools

import jax
import jax.numpy as jnp
from jax import lax
from jax.experimental import pallas as pl
from jax.experimental.pallas import tpu as pltpu
from jax.experimental.pallas import tpu_sc as plsc

B, N, C = 8, 20000, 80
NC, NS = 2, 16
NW = NC * NS
CH = 80
L = 16


def _sc_gather(bcol, ccol, ncol, boxes_flat, scores_flat, *, kp):
    p = kp // NW
    t_sub = p // CH

    mesh = plsc.VectorSubcoreMesh(
        core_axis_name="c", subcore_axis_name="s",
        num_cores=NC, num_subcores=NS)

    @functools.partial(
        pl.kernel,
        mesh=mesh,
        compiler_params=pltpu.CompilerParams(use_tc_tiling_on_sc=False),
        out_type=(
            jax.ShapeDtypeStruct((4, kp), jnp.float32),
            jax.ShapeDtypeStruct((kp,), jnp.float32),
        ),
        scratch_types=[
            pltpu.VMEM((p,), jnp.int32),
            pltpu.VMEM((p,), jnp.int32),
            pltpu.VMEM((p,), jnp.int32),
            pltpu.VMEM((4, t_sub, CH), jnp.int32),
            pltpu.VMEM((t_sub, CH), jnp.int32),
            pltpu.VMEM((4, p), jnp.float32),
            pltpu.VMEM((p,), jnp.float32),
            pltpu.SemaphoreType.DMA,
        ],
    )
    def body(bcol_h, ccol_h, ncol_h, boxes_h, scores_h,
             boxes_out_h, scores_out_h,
             bcol_v, ccol_v, ncol_v, bidx_v, sidx_v, bcols_v, srows_v, sem):
        wid = lax.axis_index("s") * NC + lax.axis_index("c")
        base = wid * p
        pltpu.sync_copy(bcol_h.at[pl.ds(base, p)], bcol_v)
        pltpu.sync_copy(ccol_h.at[pl.ds(base, p)], ccol_v)
        pltpu.sync_copy(ncol_h.at[pl.ds(base, p)], ncol_v)

        for t in range(t_sub):
            for j in range(CH // L):
                src = pl.ds(t * CH + j * L, L)
                b = bcol_v[src]
                c = ccol_v[src]
                n = ncol_v[src]
                bn4 = (b * N + n) * 4
                for col in range(4):
                    bidx_v[col, t, pl.ds(j * L, L)] = bn4 + col
                sidx_v[t, pl.ds(j * L, L)] = (b * C + c) * N + n

        copies = []
        for t in range(t_sub):
            for col in range(4):
                copies.append(pltpu.async_copy(
                    boxes_h.at[bidx_v.at[col, t]],
                    bcols_v.at[col, pl.ds(t * CH, CH)], sem))
            copies.append(pltpu.async_copy(
                scores_h.at[sidx_v.at[t]], srows_v.at[pl.ds(t * CH, CH)], sem))
        for cp in copies:
            cp.wait()

        for col in range(4):
            pltpu.sync_copy(bcols_v.at[col],
                            boxes_out_h.at[col, pl.ds(base, p)])
        pltpu.sync_copy(srows_v, scores_out_h.at[pl.ds(base, p)])

    return body(bcol, ccol, ncol, boxes_flat, scores_flat)


def kernel(boxes, scores, selected_indices):
    k = selected_indices.shape[0]
    kp = -(-k // (8 * NW)) * (8 * NW)
    sel32 = selected_indices.astype(jnp.int32)
    selp = jnp.concatenate(
        [sel32, jnp.zeros((kp - k, 3), jnp.int32)], axis=0)
    boxes_cols, scores_sel = _sc_gather(
        selp[:, 0], selp[:, 1], selp[:, 2],
        boxes.reshape(-1), scores.reshape(-1), kp=kp)
    dets = jnp.concatenate(
        [boxes_cols[:, :k].T, scores_sel[:k, None]], axis=1)
    return dets, selected_indices[:, 0], selected_indices[:, 1]
